# Initial kernel scaffold; baseline (speedup 1.0000x reference)
import jax
import jax.numpy as jnp
from jax.experimental import pallas as pl


def kernel(x, edge_index, edge_type, coeff, bases, h_bias, loop_weight):
    raise NotImplementedError("write your pallas kernel here")


# speedup vs baseline: 5.7522x; 5.7522x over previous
import functools

import jax
import jax.numpy as jnp
from jax import lax
from jax.experimental import pallas as pl
from jax.experimental.pallas import tpu as pltpu
from jax.experimental.pallas import tpu_sc as plsc

N_NODES = 10000
N_EDGES = 320000
N_REL = 8
N_BASES = 4
D = 128
DH = D // 2
RN = N_REL * N_NODES

NC, NS, L = 2, 16, 16
NW = NC * NS
EPW = N_EDGES // NW
EPT = N_EDGES // NS
G = 128
FULL_CHUNKS = EPT // G
TAIL = EPT - FULL_CHUNKS * G
ECH = 2000
NPAIR = N_NODES // 2
RPS = 312
REM = NPAIR - NS * RPS
_COPY_CHUNKS = ((0, 128), (128, 128), (256, 56))

_mesh = plsc.VectorSubcoreMesh(
    core_axis_name="c", subcore_axis_name="s", num_cores=NC, num_subcores=NS
)
_sc_params = pltpu.CompilerParams(needs_layout_passes=False)


def _xw_body(coeff_ref, bases_ref, x_ref, o_ref):
    r = pl.program_id(0)
    w = coeff_ref[r, 0] * bases_ref[0]
    for b in range(1, N_BASES):
        w = w + coeff_ref[r, b] * bases_ref[b]
    o_ref[0] = jnp.dot(x_ref[...], w, preferred_element_type=jnp.float32)


def _stage_a(coeff, bases, x):
    return pl.pallas_call(
        _xw_body,
        grid=(N_REL,),
        in_specs=[
            pl.BlockSpec(memory_space=pltpu.SMEM),
            pl.BlockSpec((N_BASES, D, D), lambda r: (0, 0, 0)),
            pl.BlockSpec((N_NODES, D), lambda r: (0, 0)),
        ],
        out_specs=pl.BlockSpec((1, N_NODES, D), lambda r: (r, 0, 0)),
        out_shape=jax.ShapeDtypeStruct((N_REL, N_NODES, D), jnp.float32),
    )(coeff, bases, x)


@functools.partial(
    pl.kernel,
    out_type=jax.ShapeDtypeStruct((NW * RN,), jnp.float32),
    mesh=_mesh,
    scratch_types=[
        pltpu.VMEM((RN,), jnp.float32),
        pltpu.VMEM((ECH,), jnp.int32),
        pltpu.VMEM((ECH,), jnp.int32),
    ],
    compiler_params=_sc_params,
)
def _deg_kernel(dst_hbm, et_hbm, out_hbm, deg_v, d_buf, t_buf):
    cid = lax.axis_index("c")
    sid = lax.axis_index("s")
    wid = sid * NC + cid
    base = wid * EPW

    zero16 = jnp.zeros((L,), jnp.float32)

    def zbody(i, carry):
        for u in range(8):
            deg_v[pl.ds(i * 128 + u * L, L)] = zero16
        return carry

    lax.fori_loop(0, RN // 128, zbody, 0)

    one16 = jnp.full((L,), 1.0, jnp.float32)

    def cbody(c, carry):
        off = base + c * ECH
        pltpu.sync_copy(dst_hbm.at[pl.ds(off, ECH)], d_buf)
        pltpu.sync_copy(et_hbm.at[pl.ds(off, ECH)], t_buf)

        def ebody(j, inner):
            d16 = d_buf[pl.ds(j * L, L)]
            t16 = t_buf[pl.ds(j * L, L)]
            plsc.addupdate_scatter(deg_v, [t16 * N_NODES + d16], one16)
            return inner

        lax.fori_loop(0, ECH // L, ebody, 0)
        return carry

    lax.fori_loop(0, EPW // ECH, cbody, 0)
    pltpu.sync_copy(deg_v, out_hbm.at[pl.ds(wid * RN, RN)])


def _inv_body(d_ref, o_ref):
    s = jnp.sum(d_ref[...], axis=0)
    o_ref[...] = 1.0 / jnp.maximum(s, 1.0)


def _stage_c(deg_all):
    d3 = deg_all.reshape(NW, RN // D, D)
    out = pl.pallas_call(
        _inv_body,
        out_shape=jax.ShapeDtypeStruct((RN // D, D), jnp.float32),
    )(d3)
    return out.reshape(RN)


@functools.partial(
    pl.kernel,
    out_type=jax.ShapeDtypeStruct((N_EDGES,), jnp.float32),
    mesh=_mesh,
    scratch_types=[
        pltpu.VMEM((RN,), jnp.float32),
        pltpu.VMEM((ECH,), jnp.int32),
        pltpu.VMEM((ECH,), jnp.int32),
        pltpu.VMEM((ECH,), jnp.float32),
    ],
    compiler_params=_sc_params,
)
def _scale_kernel(inv_hbm, dst_hbm, et_hbm, out_hbm, inv_v, d_buf, t_buf,
                  s_buf):
    cid = lax.axis_index("c")
    sid = lax.axis_index("s")
    wid = sid * NC + cid
    base = wid * EPW

    pltpu.sync_copy(inv_hbm, inv_v)

    def cbody(c, carry):
        off = base + c * ECH
        pltpu.sync_copy(dst_hbm.at[pl.ds(off, ECH)], d_buf)
        pltpu.sync_copy(et_hbm.at[pl.ds(off, ECH)], t_buf)

        def ebody(j, inner):
            d16 = d_buf[pl.ds(j * L, L)]
            t16 = t_buf[pl.ds(j * L, L)]
            s_buf[pl.ds(j * L, L)] = plsc.load_gather(
                inv_v, [t16 * N_NODES + d16])
            return inner

        lax.fori_loop(0, ECH // L, ebody, 0)
        pltpu.sync_copy(s_buf, out_hbm.at[pl.ds(off, ECH)])
        return carry

    lax.fori_loop(0, EPW // ECH, cbody, 0)


@functools.partial(
    pl.kernel,
    out_type=jax.ShapeDtypeStruct((NC, NPAIR, D), jnp.float32),
    mesh=_mesh,
    scratch_types=[
        pltpu.VMEM_SHARED((NPAIR, D), jnp.float32),
        pltpu.VMEM((G, D), jnp.float32),
        pltpu.VMEM((G, D), jnp.float32),
        pltpu.VMEM((G,), jnp.int32),
        pltpu.VMEM((G,), jnp.int32),
        pltpu.VMEM((G,), jnp.int32),
        pltpu.VMEM((G,), jnp.int32),
        pltpu.VMEM((G,), jnp.int32),
        pltpu.VMEM((G,), jnp.float32),
        pltpu.VMEM((TAIL, D), jnp.float32),
        pltpu.VMEM((TAIL, D), jnp.float32),
        pltpu.VMEM((TAIL,), jnp.int32),
        pltpu.VMEM((TAIL,), jnp.int32),
        pltpu.VMEM((TAIL,), jnp.int32),
        pltpu.VMEM((TAIL,), jnp.int32),
        pltpu.VMEM((TAIL,), jnp.int32),
        pltpu.VMEM((TAIL,), jnp.float32),
        pltpu.SemaphoreType.DMA,
    ],
    compiler_params=_sc_params,
)
def _scatter_kernel(xw_hbm, s_hbm, src_hbm, dst_hbm, et_hbm, out_hbm,
                    acc_sh, rows, pbuf, srcb, dstb, etb, gidx, didx,
                    sbuf, rows_t, pbuf_t, srcb_t, dstb_t, etb_t, gidx_t,
                    didx_t, sbuf_t, sem):
    cid = lax.axis_index("c")
    sid = lax.axis_index("s")
    base = sid * EPT
    col0 = cid * DH

    zero16 = jnp.zeros((L,), jnp.float32)

    def zb(i, carry):
        for u in range(D // L):
            rows[i, pl.ds(u * L, L)] = zero16
        return carry

    lax.fori_loop(0, G, zb, 0)
    row0 = sid * RPS
    for k, nrows in _COPY_CHUNKS:
        pltpu.sync_copy(rows.at[pl.ds(0, nrows)], acc_sh.at[pl.ds(row0 + k, nrows)])

    @pl.when(sid == 0)
    def _zero_rem():
        pltpu.sync_copy(rows.at[pl.ds(0, REM)], acc_sh.at[pl.ds(NS * RPS, REM)])

    plsc.subcore_barrier()

    def _scale_pack(rows_ref, p_ref, s_ref, d_ref, n_edges):
        def mbody(mb, carry):
            s16 = s_ref[pl.ds(mb * L, L)]
            par16 = (d_ref[pl.ds(mb * L, L)] & 1) * DH
            for mm in range(L):
                m = mb * L + mm
                sv = s16[mm]
                pv = par16[mm]
                nv = DH - pv
                for q in range(DH // L):
                    p_ref[m, pl.ds(pv + q * L, L)] = (
                        rows_ref[m, pl.ds(col0 + q * L, L)] * sv)
                    p_ref[m, pl.ds(nv + q * L, L)] = zero16
            return carry

        lax.fori_loop(0, n_edges // L, mbody, 0)

    def chunk(c, carry):
        off = base + c * G
        pltpu.sync_copy(src_hbm.at[pl.ds(off, G)], srcb)
        pltpu.sync_copy(dst_hbm.at[pl.ds(off, G)], dstb)
        pltpu.sync_copy(et_hbm.at[pl.ds(off, G)], etb)
        pltpu.sync_copy(s_hbm.at[pl.ds(off, G)], sbuf)
        for j in range(G // L):
            s16 = srcb[pl.ds(j * L, L)]
            t16 = etb[pl.ds(j * L, L)]
            d16 = dstb[pl.ds(j * L, L)]
            gidx[pl.ds(j * L, L)] = t16 * N_NODES + s16
            didx[pl.ds(j * L, L)] = d16 >> 1
        pltpu.async_copy(xw_hbm.at[gidx], rows, sem).wait()
        _scale_pack(rows, pbuf, sbuf, dstb, G)
        pltpu.sync_copy(pbuf, acc_sh.at[didx], add=True)
        return carry

    lax.fori_loop(0, FULL_CHUNKS, chunk, 0)

    off = base + FULL_CHUNKS * G
    pltpu.sync_copy(src_hbm.at[pl.ds(off, TAIL)], srcb_t)
    pltpu.sync_copy(dst_hbm.at[pl.ds(off, TAIL)], dstb_t)
    pltpu.sync_copy(et_hbm.at[pl.ds(off, TAIL)], etb_t)
    pltpu.sync_copy(s_hbm.at[pl.ds(off, TAIL)], sbuf_t)
    for j in range(TAIL // L):
        s16 = srcb_t[pl.ds(j * L, L)]
        t16 = etb_t[pl.ds(j * L, L)]
        d16 = dstb_t[pl.ds(j * L, L)]
        gidx_t[pl.ds(j * L, L)] = t16 * N_NODES + s16
        didx_t[pl.ds(j * L, L)] = d16 >> 1
    pltpu.async_copy(xw_hbm.at[gidx_t], rows_t, sem).wait()
    _scale_pack(rows_t, pbuf_t, sbuf_t, dstb_t, TAIL)
    pltpu.sync_copy(pbuf_t, acc_sh.at[didx_t], add=True)

    plsc.subcore_barrier()
    for k, nrows in _COPY_CHUNKS:
        pltpu.sync_copy(acc_sh.at[pl.ds(row0 + k, nrows)], rows.at[pl.ds(0, nrows)])
        pltpu.sync_copy(rows.at[pl.ds(0, nrows)],
                        out_hbm.at[cid, pl.ds(row0 + k, nrows)])

    @pl.when(sid == 0)
    def _write_rem():
        pltpu.sync_copy(acc_sh.at[pl.ds(NS * RPS, REM)], rows.at[pl.ds(0, REM)])
        pltpu.sync_copy(rows.at[pl.ds(0, REM)],
                        out_hbm.at[cid, pl.ds(NS * RPS, REM)])


def _fin_body(hp_ref, x_ref, lw_ref, b_ref, o_ref):
    o_ref[...] = (jnp.concatenate([hp_ref[0], hp_ref[1]], axis=1)
                  + jnp.dot(x_ref[...], lw_ref[...],
                            preferred_element_type=jnp.float32)
                  + b_ref[...])


def _stage_e(hp, x, loop_weight, h_bias):
    return pl.pallas_call(
        _fin_body,
        out_shape=jax.ShapeDtypeStruct((N_NODES, D), jnp.float32),
    )(hp, x, loop_weight, h_bias.reshape(1, D))


@jax.jit
def kernel(x, edge_index, edge_type, coeff, bases, h_bias, loop_weight):
    src = edge_index[0].astype(jnp.int32)
    dst = edge_index[1].astype(jnp.int32)
    et = edge_type.astype(jnp.int32)

    xw = _stage_a(coeff, bases, x).reshape(RN, D)
    deg_all = _deg_kernel(dst, et)
    inv_deg = _stage_c(deg_all)
    scales = _scale_kernel(inv_deg, dst, et)
    hp = _scatter_kernel(xw, scales, src, dst, et)
    return _stage_e(hp.reshape(NC, N_NODES, DH), x, loop_weight, h_bias)


# speedup vs baseline: 10.6974x; 1.8597x over previous
import functools

import jax
import jax.numpy as jnp
from jax import lax
from jax.experimental import pallas as pl
from jax.experimental.pallas import tpu as pltpu
from jax.experimental.pallas import tpu_sc as plsc

N_NODES = 10000
N_EDGES = 320000
N_REL = 8
N_BASES = 4
D = 128
DH = D // 2
RN = N_REL * N_NODES

NC, NS, L = 2, 16, 16
NW = NC * NS
EPW = N_EDGES // NW
EPT = N_EDGES // NS
G2 = 64
SCH = 1536
CPS = SCH // G2
NSCH = 13
TAIL = EPT - NSCH * SCH
ECH = 2000
NPAIR = N_NODES // 2
RPS = 312
REM = NPAIR - NS * RPS
_COPY_CHUNKS = ((0, 64), (64, 64), (128, 64), (192, 64), (256, 56))

_mesh = plsc.VectorSubcoreMesh(
    core_axis_name="c", subcore_axis_name="s", num_cores=NC, num_subcores=NS
)
_sc_params = pltpu.CompilerParams(needs_layout_passes=False)


def _xw_body(coeff_ref, bases_ref, x_ref, o_ref):
    r = pl.program_id(0)
    w = coeff_ref[r, 0] * bases_ref[0]
    for b in range(1, N_BASES):
        w = w + coeff_ref[r, b] * bases_ref[b]
    o_ref[0] = jnp.dot(x_ref[...], w, preferred_element_type=jnp.float32)


def _stage_a(coeff, bases, x):
    return pl.pallas_call(
        _xw_body,
        grid=(N_REL,),
        in_specs=[
            pl.BlockSpec(memory_space=pltpu.SMEM),
            pl.BlockSpec((N_BASES, D, D), lambda r: (0, 0, 0)),
            pl.BlockSpec((N_NODES, D), lambda r: (0, 0)),
        ],
        out_specs=pl.BlockSpec((1, N_NODES, D), lambda r: (r, 0, 0)),
        out_shape=jax.ShapeDtypeStruct((N_REL, N_NODES, D), jnp.float32),
    )(coeff, bases, x)


@functools.partial(
    pl.kernel,
    out_type=jax.ShapeDtypeStruct((NW * RN,), jnp.float32),
    mesh=_mesh,
    scratch_types=[
        pltpu.VMEM((RN,), jnp.float32),
        pltpu.VMEM((ECH,), jnp.int32),
        pltpu.VMEM((ECH,), jnp.int32),
    ],
    compiler_params=_sc_params,
)
def _deg_kernel(dst_hbm, et_hbm, out_hbm, deg_v, d_buf, t_buf):
    cid = lax.axis_index("c")
    sid = lax.axis_index("s")
    wid = sid * NC + cid
    base = wid * EPW

    zero16 = jnp.zeros((L,), jnp.float32)

    def zbody(i, carry):
        for u in range(8):
            deg_v[pl.ds(i * 128 + u * L, L)] = zero16
        return carry

    lax.fori_loop(0, RN // 128, zbody, 0)

    one16 = jnp.full((L,), 1.0, jnp.float32)

    def cbody(c, carry):
        off = base + c * ECH
        pltpu.sync_copy(dst_hbm.at[pl.ds(off, ECH)], d_buf)
        pltpu.sync_copy(et_hbm.at[pl.ds(off, ECH)], t_buf)

        def ebody(j, inner):
            d16 = d_buf[pl.ds(j * L, L)]
            t16 = t_buf[pl.ds(j * L, L)]
            plsc.addupdate_scatter(deg_v, [t16 * N_NODES + d16], one16)
            return inner

        lax.fori_loop(0, ECH // L, ebody, 0)
        return carry

    lax.fori_loop(0, EPW // ECH, cbody, 0)
    pltpu.sync_copy(deg_v, out_hbm.at[pl.ds(wid * RN, RN)])


def _inv_body(d_ref, o_ref):
    s = jnp.sum(d_ref[...], axis=0)
    o_ref[...] = 1.0 / jnp.maximum(s, 1.0)


def _stage_c(deg_all):
    d3 = deg_all.reshape(NW, RN // D, D)
    out = pl.pallas_call(
        _inv_body,
        out_shape=jax.ShapeDtypeStruct((RN // D, D), jnp.float32),
    )(d3)
    return out.reshape(RN)


@functools.partial(
    pl.kernel,
    out_type=jax.ShapeDtypeStruct((N_EDGES,), jnp.float32),
    mesh=_mesh,
    scratch_types=[
        pltpu.VMEM((RN,), jnp.float32),
        pltpu.VMEM((ECH,), jnp.int32),
        pltpu.VMEM((ECH,), jnp.int32),
        pltpu.VMEM((ECH,), jnp.float32),
    ],
    compiler_params=_sc_params,
)
def _scale_kernel(inv_hbm, dst_hbm, et_hbm, out_hbm, inv_v, d_buf, t_buf,
                  s_buf):
    cid = lax.axis_index("c")
    sid = lax.axis_index("s")
    wid = sid * NC + cid
    base = wid * EPW

    pltpu.sync_copy(inv_hbm, inv_v)

    def cbody(c, carry):
        off = base + c * ECH
        pltpu.sync_copy(dst_hbm.at[pl.ds(off, ECH)], d_buf)
        pltpu.sync_copy(et_hbm.at[pl.ds(off, ECH)], t_buf)

        def ebody(j, inner):
            d16 = d_buf[pl.ds(j * L, L)]
            t16 = t_buf[pl.ds(j * L, L)]
            s_buf[pl.ds(j * L, L)] = plsc.load_gather(
                inv_v, [t16 * N_NODES + d16])
            return inner

        lax.fori_loop(0, ECH // L, ebody, 0)
        pltpu.sync_copy(s_buf, out_hbm.at[pl.ds(off, ECH)])
        return carry

    lax.fori_loop(0, EPW // ECH, cbody, 0)


@functools.partial(
    pl.kernel,
    out_type=jax.ShapeDtypeStruct((NC, NPAIR, D), jnp.float32),
    mesh=_mesh,
    scratch_types=[
        pltpu.VMEM_SHARED((NPAIR, D), jnp.float32),
        pltpu.VMEM((G2, D), jnp.float32),
        pltpu.VMEM((G2, D), jnp.float32),
        pltpu.VMEM((G2, D), jnp.float32),
        pltpu.VMEM((G2, D), jnp.float32),
        pltpu.VMEM((SCH,), jnp.int32),
        pltpu.VMEM((SCH,), jnp.int32),
        pltpu.VMEM((SCH,), jnp.int32),
        pltpu.VMEM((SCH,), jnp.float32),
        pltpu.VMEM((CPS, 1, G2), jnp.int32),
        pltpu.VMEM((CPS, 1, G2), jnp.int32),
        pltpu.VMEM((TAIL,), jnp.int32),
        pltpu.VMEM((TAIL,), jnp.int32),
        pltpu.SemaphoreType.DMA,
        pltpu.SemaphoreType.DMA,
        pltpu.SemaphoreType.DMA,
        pltpu.SemaphoreType.DMA,
    ],
    compiler_params=_sc_params,
)
def _scatter_kernel(xw_hbm, s_hbm, src_hbm, dst_hbm, et_hbm, out_hbm,
                    acc_sh, rows0, rows1, pbuf0, pbuf1,
                    srcm, dstm, etm, sm, gkeys, dkeys, gidx_t, didx_t,
                    gsem0, gsem1, ssem0, ssem1):
    cid = lax.axis_index("c")
    sid = lax.axis_index("s")
    base = sid * EPT
    col0 = cid * DH

    zero16 = jnp.zeros((L,), jnp.float32)

    def zb(i, carry):
        for u in range(D // L):
            pbuf0[i, pl.ds(u * L, L)] = zero16
        return carry

    lax.fori_loop(0, G2, zb, 0)
    row0 = sid * RPS
    for k, nrows in _COPY_CHUNKS:
        pltpu.sync_copy(pbuf0.at[pl.ds(0, nrows)], acc_sh.at[pl.ds(row0 + k, nrows)])

    @pl.when(sid == 0)
    def _zero_rem():
        pltpu.sync_copy(pbuf0.at[pl.ds(0, REM)], acc_sh.at[pl.ds(NS * RPS, REM)])

    plsc.subcore_barrier()

    def _scale_pack(rows_ref, p_ref, eoff, n_edges):
        def mbody(mb, carry):
            o = eoff + mb * L
            s16 = sm[pl.ds(o, L)]
            par16 = (dstm[pl.ds(o, L)] & 1) * DH
            for mm in range(L):
                m = mb * L + mm
                sv = s16[mm]
                pv = par16[mm]
                nv = DH - pv
                for q in range(DH // L):
                    p_ref[m, pl.ds(pv + q * L, L)] = (
                        rows_ref[m, pl.ds(col0 + q * L, L)] * sv)
                    p_ref[m, pl.ds(nv + q * L, L)] = zero16
            return carry

        lax.fori_loop(0, n_edges // L, mbody, 0)

    def keys_body(c2, carry):
        for j in range(G2 // L):
            o = c2 * G2 + j * L
            s16 = srcm[pl.ds(o, L)]
            t16 = etm[pl.ds(o, L)]
            d16 = dstm[pl.ds(o, L)]
            gkeys[c2, 0, pl.ds(j * L, L)] = t16 * N_NODES + s16
            dkeys[c2, 0, pl.ds(j * L, L)] = d16 >> 1
        return carry

    def superchunk(sc, carry):
        off = base + sc * SCH
        pltpu.sync_copy(src_hbm.at[pl.ds(off, SCH)], srcm)
        pltpu.sync_copy(dst_hbm.at[pl.ds(off, SCH)], dstm)
        pltpu.sync_copy(et_hbm.at[pl.ds(off, SCH)], etm)
        pltpu.sync_copy(s_hbm.at[pl.ds(off, SCH)], sm)
        lax.fori_loop(0, CPS, keys_body, 0)

        pltpu.async_copy(xw_hbm.at[gkeys.at[0, 0]], rows0, gsem0)

        def pairbody(p, inner):
            c0 = 2 * p
            pltpu.make_async_copy(xw_hbm.at[gkeys.at[0, 0]], rows0, gsem0).wait()
            pltpu.async_copy(xw_hbm.at[gkeys.at[c0 + 1, 0]], rows1, gsem1)

            @pl.when(p > 0)
            def _drain_s0():
                pltpu.make_async_copy(
                    pbuf0, acc_sh.at[dkeys.at[0, 0]], ssem0).wait()

            _scale_pack(rows0, pbuf0, c0 * G2, G2)
            pltpu.async_copy(pbuf0, acc_sh.at[dkeys.at[c0, 0]], ssem0, add=True)

            pltpu.make_async_copy(xw_hbm.at[gkeys.at[0, 0]], rows1, gsem1).wait()

            @pl.when(p < CPS // 2 - 1)
            def _prefetch():
                pltpu.async_copy(xw_hbm.at[gkeys.at[c0 + 2, 0]], rows0, gsem0)

            @pl.when(p > 0)
            def _drain_s1():
                pltpu.make_async_copy(
                    pbuf1, acc_sh.at[dkeys.at[0, 0]], ssem1).wait()

            _scale_pack(rows1, pbuf1, (c0 + 1) * G2, G2)
            pltpu.async_copy(pbuf1, acc_sh.at[dkeys.at[c0 + 1, 0]], ssem1,
                             add=True)
            return inner

        lax.fori_loop(0, CPS // 2, pairbody, 0)
        pltpu.make_async_copy(pbuf0, acc_sh.at[dkeys.at[0, 0]], ssem0).wait()
        pltpu.make_async_copy(pbuf1, acc_sh.at[dkeys.at[0, 0]], ssem1).wait()
        return carry

    lax.fori_loop(0, NSCH, superchunk, 0)

    off = base + NSCH * SCH
    pltpu.sync_copy(src_hbm.at[pl.ds(off, TAIL)], srcm.at[pl.ds(0, TAIL)])
    pltpu.sync_copy(dst_hbm.at[pl.ds(off, TAIL)], dstm.at[pl.ds(0, TAIL)])
    pltpu.sync_copy(et_hbm.at[pl.ds(off, TAIL)], etm.at[pl.ds(0, TAIL)])
    pltpu.sync_copy(s_hbm.at[pl.ds(off, TAIL)], sm.at[pl.ds(0, TAIL)])
    for j in range(TAIL // L):
        s16 = srcm[pl.ds(j * L, L)]
        t16 = etm[pl.ds(j * L, L)]
        d16 = dstm[pl.ds(j * L, L)]
        gidx_t[pl.ds(j * L, L)] = t16 * N_NODES + s16
        didx_t[pl.ds(j * L, L)] = d16 >> 1
    pltpu.async_copy(xw_hbm.at[gidx_t], rows0.at[pl.ds(0, TAIL)], gsem0).wait()
    _scale_pack(rows0, pbuf0, 0, TAIL)
    pltpu.sync_copy(pbuf0.at[pl.ds(0, TAIL)], acc_sh.at[didx_t], add=True)

    plsc.subcore_barrier()
    for k, nrows in _COPY_CHUNKS:
        pltpu.sync_copy(acc_sh.at[pl.ds(row0 + k, nrows)], pbuf0.at[pl.ds(0, nrows)])
        pltpu.sync_copy(pbuf0.at[pl.ds(0, nrows)],
                        out_hbm.at[cid, pl.ds(row0 + k, nrows)])

    @pl.when(sid == 0)
    def _write_rem():
        pltpu.sync_copy(acc_sh.at[pl.ds(NS * RPS, REM)], pbuf0.at[pl.ds(0, REM)])
        pltpu.sync_copy(pbuf0.at[pl.ds(0, REM)],
                        out_hbm.at[cid, pl.ds(NS * RPS, REM)])


def _fin_body(hp_ref, x_ref, lw_ref, b_ref, o_ref):
    o_ref[...] = (jnp.concatenate([hp_ref[0], hp_ref[1]], axis=1)
                  + jnp.dot(x_ref[...], lw_ref[...],
                            preferred_element_type=jnp.float32)
                  + b_ref[...])


def _stage_e(hp, x, loop_weight, h_bias):
    return pl.pallas_call(
        _fin_body,
        out_shape=jax.ShapeDtypeStruct((N_NODES, D), jnp.float32),
    )(hp, x, loop_weight, h_bias.reshape(1, D))


@jax.jit
def kernel(x, edge_index, edge_type, coeff, bases, h_bias, loop_weight):
    src = edge_index[0].astype(jnp.int32)
    dst = edge_index[1].astype(jnp.int32)
    et = edge_type.astype(jnp.int32)

    xw = _stage_a(coeff, bases, x).reshape(RN, D)
    deg_all = _deg_kernel(dst, et)
    inv_deg = _stage_c(deg_all)
    scales = _scale_kernel(inv_deg, dst, et)
    hp = _scatter_kernel(xw, scales, src, dst, et)
    return _stage_e(hp.reshape(NC, N_NODES, DH), x, loop_weight, h_bias)


# speedup vs baseline: 11.0245x; 1.0306x over previous
import functools

import jax
import jax.numpy as jnp
from jax import lax
from jax.experimental import pallas as pl
from jax.experimental.pallas import tpu as pltpu
from jax.experimental.pallas import tpu_sc as plsc

N_NODES = 10000
N_EDGES = 320000
N_REL = 8
N_BASES = 4
D = 128
DH = D // 2
RN = N_REL * N_NODES

NC, NS, L = 2, 16, 16
NW = NC * NS
EPW = N_EDGES // NW
EPT = N_EDGES // NS
G2 = 64
SCH = 1536
CPS = SCH // G2
NSCH = 13
TAIL = EPT - NSCH * SCH
ECH = 2000
NPAIR = N_NODES // 2
RPS = 312
REM = NPAIR - NS * RPS
_COPY_CHUNKS = ((0, 64), (64, 64), (128, 64), (192, 64), (256, 56))

_mesh = plsc.VectorSubcoreMesh(
    core_axis_name="c", subcore_axis_name="s", num_cores=NC, num_subcores=NS
)
_sc_params = pltpu.CompilerParams(needs_layout_passes=False)


def _xw_body(coeff_ref, bases_ref, x_ref, o_ref):
    r = pl.program_id(0)
    w = coeff_ref[r, 0] * bases_ref[0]
    for b in range(1, N_BASES):
        w = w + coeff_ref[r, b] * bases_ref[b]
    o_ref[0] = jnp.dot(x_ref[...], w, preferred_element_type=jnp.float32)


def _stage_a(coeff, bases, x):
    return pl.pallas_call(
        _xw_body,
        grid=(N_REL,),
        in_specs=[
            pl.BlockSpec(memory_space=pltpu.SMEM),
            pl.BlockSpec((N_BASES, D, D), lambda r: (0, 0, 0)),
            pl.BlockSpec((N_NODES, D), lambda r: (0, 0)),
        ],
        out_specs=pl.BlockSpec((1, N_NODES, D), lambda r: (r, 0, 0)),
        out_shape=jax.ShapeDtypeStruct((N_REL, N_NODES, D), jnp.float32),
    )(coeff, bases, x)


@functools.partial(
    pl.kernel,
    out_type=jax.ShapeDtypeStruct((NW * RN,), jnp.float32),
    mesh=_mesh,
    scratch_types=[
        pltpu.VMEM((RN,), jnp.float32),
        pltpu.VMEM((ECH,), jnp.int32),
        pltpu.VMEM((ECH,), jnp.int32),
    ],
    compiler_params=_sc_params,
)
def _deg_kernel(dst_hbm, et_hbm, out_hbm, deg_v, d_buf, t_buf):
    cid = lax.axis_index("c")
    sid = lax.axis_index("s")
    wid = sid * NC + cid
    base = wid * EPW

    zero16 = jnp.zeros((L,), jnp.float32)

    def zbody(i, carry):
        for u in range(8):
            deg_v[pl.ds(i * 128 + u * L, L)] = zero16
        return carry

    lax.fori_loop(0, RN // 128, zbody, 0)

    one16 = jnp.full((L,), 1.0, jnp.float32)

    def cbody(c, carry):
        off = base + c * ECH
        pltpu.sync_copy(dst_hbm.at[pl.ds(off, ECH)], d_buf)
        pltpu.sync_copy(et_hbm.at[pl.ds(off, ECH)], t_buf)

        def ebody(j, inner):
            d16 = d_buf[pl.ds(j * L, L)]
            t16 = t_buf[pl.ds(j * L, L)]
            plsc.addupdate_scatter(deg_v, [t16 * N_NODES + d16], one16)
            return inner

        lax.fori_loop(0, ECH // L, ebody, 0)
        return carry

    lax.fori_loop(0, EPW // ECH, cbody, 0)
    pltpu.sync_copy(deg_v, out_hbm.at[pl.ds(wid * RN, RN)])


def _inv_body(d_ref, o_ref):
    s = jnp.sum(d_ref[...], axis=0)
    o_ref[...] = 1.0 / jnp.maximum(s, 1.0)


def _stage_c(deg_all):
    d3 = deg_all.reshape(NW, RN // D, D)
    out = pl.pallas_call(
        _inv_body,
        out_shape=jax.ShapeDtypeStruct((RN // D, D), jnp.float32),
    )(d3)
    return out.reshape(RN)


@functools.partial(
    pl.kernel,
    out_type=jax.ShapeDtypeStruct((N_EDGES,), jnp.float32),
    mesh=_mesh,
    scratch_types=[
        pltpu.VMEM((RN,), jnp.float32),
        pltpu.VMEM((ECH,), jnp.int32),
        pltpu.VMEM((ECH,), jnp.int32),
        pltpu.VMEM((ECH,), jnp.float32),
    ],
    compiler_params=_sc_params,
)
def _scale_kernel(inv_hbm, dst_hbm, et_hbm, out_hbm, inv_v, d_buf, t_buf,
                  s_buf):
    cid = lax.axis_index("c")
    sid = lax.axis_index("s")
    wid = sid * NC + cid
    base = wid * EPW

    pltpu.sync_copy(inv_hbm, inv_v)

    def cbody(c, carry):
        off = base + c * ECH
        pltpu.sync_copy(dst_hbm.at[pl.ds(off, ECH)], d_buf)
        pltpu.sync_copy(et_hbm.at[pl.ds(off, ECH)], t_buf)

        def ebody(j, inner):
            d16 = d_buf[pl.ds(j * L, L)]
            t16 = t_buf[pl.ds(j * L, L)]
            s_buf[pl.ds(j * L, L)] = plsc.load_gather(
                inv_v, [t16 * N_NODES + d16])
            return inner

        lax.fori_loop(0, ECH // L, ebody, 0)
        pltpu.sync_copy(s_buf, out_hbm.at[pl.ds(off, ECH)])
        return carry

    lax.fori_loop(0, EPW // ECH, cbody, 0)


@functools.partial(
    pl.kernel,
    out_type=jax.ShapeDtypeStruct((NC, NPAIR, D), jnp.float32),
    mesh=_mesh,
    scratch_types=[
        pltpu.VMEM_SHARED((NPAIR, D), jnp.float32),
        pltpu.VMEM((G2, D), jnp.float32),
        pltpu.VMEM((G2, D), jnp.float32),
        pltpu.VMEM((G2, D), jnp.float32),
        pltpu.VMEM((G2, D), jnp.float32),
        pltpu.VMEM((SCH,), jnp.int32),
        pltpu.VMEM((SCH,), jnp.int32),
        pltpu.VMEM((SCH,), jnp.int32),
        pltpu.VMEM((SCH,), jnp.float32),
        pltpu.VMEM((CPS, 1, G2), jnp.int32),
        pltpu.VMEM((CPS, 1, G2), jnp.int32),
        pltpu.VMEM((TAIL,), jnp.int32),
        pltpu.VMEM((TAIL,), jnp.int32),
        pltpu.SemaphoreType.DMA,
        pltpu.SemaphoreType.DMA,
        pltpu.SemaphoreType.DMA,
        pltpu.SemaphoreType.DMA,
    ],
    compiler_params=_sc_params,
)
def _scatter_kernel(xw_hbm, s_hbm, src_hbm, dst_hbm, et_hbm, out_hbm,
                    acc_sh, rows0, rows1, pbuf0, pbuf1,
                    srcm, dstm, etm, sm, gkeys, dkeys, gidx_t, didx_t,
                    gsem0, gsem1, ssem0, ssem1):
    cid = lax.axis_index("c")
    sid = lax.axis_index("s")
    base = sid * EPT
    col0 = cid * DH

    zero16 = jnp.zeros((L,), jnp.float32)

    def zb(i, carry):
        for u in range(D // L):
            pbuf0[i, pl.ds(u * L, L)] = zero16
        return carry

    lax.fori_loop(0, G2, zb, 0)
    row0 = sid * RPS
    for k, nrows in _COPY_CHUNKS:
        pltpu.sync_copy(pbuf0.at[pl.ds(0, nrows)], acc_sh.at[pl.ds(row0 + k, nrows)])

    @pl.when(sid == 0)
    def _zero_rem():
        pltpu.sync_copy(pbuf0.at[pl.ds(0, REM)], acc_sh.at[pl.ds(NS * RPS, REM)])

    plsc.subcore_barrier()

    def _scale_pack(rows_ref, p_ref, eoff, n_edges):
        @plsc.parallel_loop(0, n_edges // L)
        def mbody(mb):
            o = eoff + mb * L
            s16 = sm[pl.ds(o, L)]
            par16 = (dstm[pl.ds(o, L)] & 1) * DH
            for mm in range(L):
                m = mb * L + mm
                sv = s16[mm]
                pv = par16[mm]
                nv = DH - pv
                for q in range(DH // L):
                    p_ref[m, pl.ds(pv + q * L, L)] = (
                        rows_ref[m, pl.ds(col0 + q * L, L)] * sv)
                    p_ref[m, pl.ds(nv + q * L, L)] = zero16

    def keys_body(c2, carry):
        for j in range(G2 // L):
            o = c2 * G2 + j * L
            s16 = srcm[pl.ds(o, L)]
            t16 = etm[pl.ds(o, L)]
            d16 = dstm[pl.ds(o, L)]
            gkeys[c2, 0, pl.ds(j * L, L)] = t16 * N_NODES + s16
            dkeys[c2, 0, pl.ds(j * L, L)] = d16 >> 1
        return carry

    def superchunk(sc, carry):
        off = base + sc * SCH
        pltpu.sync_copy(src_hbm.at[pl.ds(off, SCH)], srcm)
        pltpu.sync_copy(dst_hbm.at[pl.ds(off, SCH)], dstm)
        pltpu.sync_copy(et_hbm.at[pl.ds(off, SCH)], etm)
        pltpu.sync_copy(s_hbm.at[pl.ds(off, SCH)], sm)
        lax.fori_loop(0, CPS, keys_body, 0)

        pltpu.async_copy(xw_hbm.at[gkeys.at[0, 0]], rows0, gsem0)

        def pairbody(p, inner):
            c0 = 2 * p
            pltpu.make_async_copy(xw_hbm.at[gkeys.at[0, 0]], rows0, gsem0).wait()
            pltpu.async_copy(xw_hbm.at[gkeys.at[c0 + 1, 0]], rows1, gsem1)

            @pl.when(p > 0)
            def _drain_s0():
                pltpu.make_async_copy(
                    pbuf0, acc_sh.at[dkeys.at[0, 0]], ssem0).wait()

            _scale_pack(rows0, pbuf0, c0 * G2, G2)
            pltpu.async_copy(pbuf0, acc_sh.at[dkeys.at[c0, 0]], ssem0, add=True)

            pltpu.make_async_copy(xw_hbm.at[gkeys.at[0, 0]], rows1, gsem1).wait()

            @pl.when(p < CPS // 2 - 1)
            def _prefetch():
                pltpu.async_copy(xw_hbm.at[gkeys.at[c0 + 2, 0]], rows0, gsem0)

            @pl.when(p > 0)
            def _drain_s1():
                pltpu.make_async_copy(
                    pbuf1, acc_sh.at[dkeys.at[0, 0]], ssem1).wait()

            _scale_pack(rows1, pbuf1, (c0 + 1) * G2, G2)
            pltpu.async_copy(pbuf1, acc_sh.at[dkeys.at[c0 + 1, 0]], ssem1,
                             add=True)
            return inner

        lax.fori_loop(0, CPS // 2, pairbody, 0)
        pltpu.make_async_copy(pbuf0, acc_sh.at[dkeys.at[0, 0]], ssem0).wait()
        pltpu.make_async_copy(pbuf1, acc_sh.at[dkeys.at[0, 0]], ssem1).wait()
        return carry

    lax.fori_loop(0, NSCH, superchunk, 0)

    off = base + NSCH * SCH
    pltpu.sync_copy(src_hbm.at[pl.ds(off, TAIL)], srcm.at[pl.ds(0, TAIL)])
    pltpu.sync_copy(dst_hbm.at[pl.ds(off, TAIL)], dstm.at[pl.ds(0, TAIL)])
    pltpu.sync_copy(et_hbm.at[pl.ds(off, TAIL)], etm.at[pl.ds(0, TAIL)])
    pltpu.sync_copy(s_hbm.at[pl.ds(off, TAIL)], sm.at[pl.ds(0, TAIL)])
    for j in range(TAIL // L):
        s16 = srcm[pl.ds(j * L, L)]
        t16 = etm[pl.ds(j * L, L)]
        d16 = dstm[pl.ds(j * L, L)]
        gidx_t[pl.ds(j * L, L)] = t16 * N_NODES + s16
        didx_t[pl.ds(j * L, L)] = d16 >> 1
    pltpu.async_copy(xw_hbm.at[gidx_t], rows0.at[pl.ds(0, TAIL)], gsem0).wait()
    _scale_pack(rows0, pbuf0, 0, TAIL)
    pltpu.sync_copy(pbuf0.at[pl.ds(0, TAIL)], acc_sh.at[didx_t], add=True)

    plsc.subcore_barrier()
    for k, nrows in _COPY_CHUNKS:
        pltpu.sync_copy(acc_sh.at[pl.ds(row0 + k, nrows)], pbuf0.at[pl.ds(0, nrows)])
        pltpu.sync_copy(pbuf0.at[pl.ds(0, nrows)],
                        out_hbm.at[cid, pl.ds(row0 + k, nrows)])

    @pl.when(sid == 0)
    def _write_rem():
        pltpu.sync_copy(acc_sh.at[pl.ds(NS * RPS, REM)], pbuf0.at[pl.ds(0, REM)])
        pltpu.sync_copy(pbuf0.at[pl.ds(0, REM)],
                        out_hbm.at[cid, pl.ds(NS * RPS, REM)])


def _fin_body(hp_ref, x_ref, lw_ref, b_ref, o_ref):
    o_ref[...] = (jnp.concatenate([hp_ref[0], hp_ref[1]], axis=1)
                  + jnp.dot(x_ref[...], lw_ref[...],
                            preferred_element_type=jnp.float32)
                  + b_ref[...])


def _stage_e(hp, x, loop_weight, h_bias):
    return pl.pallas_call(
        _fin_body,
        out_shape=jax.ShapeDtypeStruct((N_NODES, D), jnp.float32),
    )(hp, x, loop_weight, h_bias.reshape(1, D))


@jax.jit
def kernel(x, edge_index, edge_type, coeff, bases, h_bias, loop_weight):
    src = edge_index[0].astype(jnp.int32)
    dst = edge_index[1].astype(jnp.int32)
    et = edge_type.astype(jnp.int32)

    xw = _stage_a(coeff, bases, x).reshape(RN, D)
    deg_all = _deg_kernel(dst, et)
    inv_deg = _stage_c(deg_all)
    scales = _scale_kernel(inv_deg, dst, et)
    hp = _scatter_kernel(xw, scales, src, dst, et)
    return _stage_e(hp.reshape(NC, N_NODES, DH), x, loop_weight, h_bias)


# speedup vs baseline: 11.2047x; 1.0163x over previous
import functools

import jax
import jax.numpy as jnp
from jax import lax
from jax.experimental import pallas as pl
from jax.experimental.pallas import tpu as pltpu
from jax.experimental.pallas import tpu_sc as plsc

N_NODES = 10000
N_EDGES = 320000
N_REL = 8
N_BASES = 4
D = 128
DH = D // 2
RN = N_REL * N_NODES

NC, NS, L = 2, 16, 16
NW = NC * NS
EPW = N_EDGES // NW
EPT = N_EDGES // NS
G2 = 64
SCH = 1536
CPS = SCH // G2
NSCH = 13
TAIL = EPT - NSCH * SCH
ECH = 10000
NPAIR = N_NODES // 2
RPS = 312
REM = NPAIR - NS * RPS
_COPY_CHUNKS = ((0, 64), (64, 64), (128, 64), (192, 64), (256, 56))

_mesh = plsc.VectorSubcoreMesh(
    core_axis_name="c", subcore_axis_name="s", num_cores=NC, num_subcores=NS
)
_sc_params = pltpu.CompilerParams(needs_layout_passes=False)


def _xw_body(coeff_ref, bases_ref, x_ref, o_ref):
    r = pl.program_id(0)
    w = coeff_ref[r, 0] * bases_ref[0]
    for b in range(1, N_BASES):
        w = w + coeff_ref[r, b] * bases_ref[b]
    o_ref[0] = jnp.dot(x_ref[...], w, preferred_element_type=jnp.float32)


def _stage_a(coeff, bases, x):
    return pl.pallas_call(
        _xw_body,
        grid=(N_REL,),
        in_specs=[
            pl.BlockSpec(memory_space=pltpu.SMEM),
            pl.BlockSpec((N_BASES, D, D), lambda r: (0, 0, 0)),
            pl.BlockSpec((N_NODES, D), lambda r: (0, 0)),
        ],
        out_specs=pl.BlockSpec((1, N_NODES, D), lambda r: (r, 0, 0)),
        out_shape=jax.ShapeDtypeStruct((N_REL, N_NODES, D), jnp.float32),
    )(coeff, bases, x)


@functools.partial(
    pl.kernel,
    out_type=jax.ShapeDtypeStruct((NW * RN,), jnp.float32),
    mesh=_mesh,
    scratch_types=[
        pltpu.VMEM((RN,), jnp.float32),
        pltpu.VMEM((ECH,), jnp.int32),
        pltpu.VMEM((ECH,), jnp.int32),
    ],
    compiler_params=_sc_params,
)
def _deg_kernel(dst_hbm, et_hbm, out_hbm, deg_v, d_buf, t_buf):
    cid = lax.axis_index("c")
    sid = lax.axis_index("s")
    wid = sid * NC + cid
    base = wid * EPW

    zero16 = jnp.zeros((L,), jnp.float32)

    def zbody(i, carry):
        for u in range(8):
            deg_v[pl.ds(i * 128 + u * L, L)] = zero16
        return carry

    lax.fori_loop(0, RN // 128, zbody, 0)

    one16 = jnp.full((L,), 1.0, jnp.float32)

    def cbody(c, carry):
        off = base + c * ECH
        pltpu.sync_copy(dst_hbm.at[pl.ds(off, ECH)], d_buf)
        pltpu.sync_copy(et_hbm.at[pl.ds(off, ECH)], t_buf)

        def ebody(j, inner):
            d16 = d_buf[pl.ds(j * L, L)]
            t16 = t_buf[pl.ds(j * L, L)]
            plsc.addupdate_scatter(deg_v, [t16 * N_NODES + d16], one16)
            return inner

        lax.fori_loop(0, ECH // L, ebody, 0)
        return carry

    lax.fori_loop(0, EPW // ECH, cbody, 0)
    pltpu.sync_copy(deg_v, out_hbm.at[pl.ds(wid * RN, RN)])


def _inv_body(d_ref, o_ref):
    s = jnp.sum(d_ref[...], axis=0)
    o_ref[...] = 1.0 / jnp.maximum(s, 1.0)


def _stage_c(deg_all):
    d3 = deg_all.reshape(NW, RN // D, D)
    out = pl.pallas_call(
        _inv_body,
        out_shape=jax.ShapeDtypeStruct((RN // D, D), jnp.float32),
    )(d3)
    return out.reshape(RN)


@functools.partial(
    pl.kernel,
    out_type=jax.ShapeDtypeStruct((N_EDGES,), jnp.float32),
    mesh=_mesh,
    scratch_types=[
        pltpu.VMEM((RN,), jnp.float32),
        pltpu.VMEM((ECH,), jnp.int32),
        pltpu.VMEM((ECH,), jnp.int32),
        pltpu.VMEM((ECH,), jnp.float32),
    ],
    compiler_params=_sc_params,
)
def _scale_kernel(inv_hbm, dst_hbm, et_hbm, out_hbm, inv_v, d_buf, t_buf,
                  s_buf):
    cid = lax.axis_index("c")
    sid = lax.axis_index("s")
    wid = sid * NC + cid
    base = wid * EPW

    pltpu.sync_copy(inv_hbm, inv_v)

    def cbody(c, carry):
        off = base + c * ECH
        pltpu.sync_copy(dst_hbm.at[pl.ds(off, ECH)], d_buf)
        pltpu.sync_copy(et_hbm.at[pl.ds(off, ECH)], t_buf)

        @plsc.parallel_loop(0, ECH // L)
        def ebody(j):
            d16 = d_buf[pl.ds(j * L, L)]
            t16 = t_buf[pl.ds(j * L, L)]
            s_buf[pl.ds(j * L, L)] = plsc.load_gather(
                inv_v, [t16 * N_NODES + d16])

        pltpu.sync_copy(s_buf, out_hbm.at[pl.ds(off, ECH)])
        return carry

    lax.fori_loop(0, EPW // ECH, cbody, 0)


@functools.partial(
    pl.kernel,
    out_type=jax.ShapeDtypeStruct((NC, NPAIR, D), jnp.float32),
    mesh=_mesh,
    scratch_types=[
        pltpu.VMEM_SHARED((NPAIR, D), jnp.float32),
        pltpu.VMEM((G2, D), jnp.float32),
        pltpu.VMEM((G2, D), jnp.float32),
        pltpu.VMEM((G2, D), jnp.float32),
        pltpu.VMEM((G2, D), jnp.float32),
        pltpu.VMEM((SCH,), jnp.int32),
        pltpu.VMEM((SCH,), jnp.int32),
        pltpu.VMEM((SCH,), jnp.int32),
        pltpu.VMEM((SCH,), jnp.float32),
        pltpu.VMEM((CPS, 1, G2), jnp.int32),
        pltpu.VMEM((CPS, 1, G2), jnp.int32),
        pltpu.VMEM((TAIL,), jnp.int32),
        pltpu.VMEM((TAIL,), jnp.int32),
        pltpu.SemaphoreType.DMA,
        pltpu.SemaphoreType.DMA,
        pltpu.SemaphoreType.DMA,
        pltpu.SemaphoreType.DMA,
    ],
    compiler_params=_sc_params,
)
def _scatter_kernel(xw_hbm, s_hbm, src_hbm, dst_hbm, et_hbm, out_hbm,
                    acc_sh, rows0, rows1, pbuf0, pbuf1,
                    srcm, dstm, etm, sm, gkeys, dkeys, gidx_t, didx_t,
                    gsem0, gsem1, ssem0, ssem1):
    cid = lax.axis_index("c")
    sid = lax.axis_index("s")
    base = sid * EPT
    col0 = cid * DH

    zero16 = jnp.zeros((L,), jnp.float32)

    def zb(i, carry):
        for u in range(D // L):
            pbuf0[i, pl.ds(u * L, L)] = zero16
        return carry

    lax.fori_loop(0, G2, zb, 0)
    row0 = sid * RPS
    for k, nrows in _COPY_CHUNKS:
        pltpu.sync_copy(pbuf0.at[pl.ds(0, nrows)], acc_sh.at[pl.ds(row0 + k, nrows)])

    @pl.when(sid == 0)
    def _zero_rem():
        pltpu.sync_copy(pbuf0.at[pl.ds(0, REM)], acc_sh.at[pl.ds(NS * RPS, REM)])

    plsc.subcore_barrier()

    def _scale_pack(rows_ref, p_ref, eoff, n_edges):
        @plsc.parallel_loop(0, n_edges // L)
        def mbody(mb):
            o = eoff + mb * L
            s16 = sm[pl.ds(o, L)]
            par16 = (dstm[pl.ds(o, L)] & 1) * DH
            for mm in range(L):
                m = mb * L + mm
                sv = s16[mm]
                pv = par16[mm]
                nv = DH - pv
                for q in range(DH // L):
                    p_ref[m, pl.ds(pv + q * L, L)] = (
                        rows_ref[m, pl.ds(col0 + q * L, L)] * sv)
                    p_ref[m, pl.ds(nv + q * L, L)] = zero16

    def keys_body(c2, carry):
        for j in range(G2 // L):
            o = c2 * G2 + j * L
            s16 = srcm[pl.ds(o, L)]
            t16 = etm[pl.ds(o, L)]
            d16 = dstm[pl.ds(o, L)]
            gkeys[c2, 0, pl.ds(j * L, L)] = t16 * N_NODES + s16
            dkeys[c2, 0, pl.ds(j * L, L)] = d16 >> 1
        return carry

    def superchunk(sc, carry):
        off = base + sc * SCH
        pltpu.sync_copy(src_hbm.at[pl.ds(off, SCH)], srcm)
        pltpu.sync_copy(dst_hbm.at[pl.ds(off, SCH)], dstm)
        pltpu.sync_copy(et_hbm.at[pl.ds(off, SCH)], etm)
        pltpu.sync_copy(s_hbm.at[pl.ds(off, SCH)], sm)
        lax.fori_loop(0, CPS, keys_body, 0)

        pltpu.async_copy(xw_hbm.at[gkeys.at[0, 0]], rows0, gsem0)

        def pairbody(p, inner):
            c0 = 2 * p
            pltpu.make_async_copy(xw_hbm.at[gkeys.at[0, 0]], rows0, gsem0).wait()
            pltpu.async_copy(xw_hbm.at[gkeys.at[c0 + 1, 0]], rows1, gsem1)

            @pl.when(p > 0)
            def _drain_s0():
                pltpu.make_async_copy(
                    pbuf0, acc_sh.at[dkeys.at[0, 0]], ssem0).wait()

            _scale_pack(rows0, pbuf0, c0 * G2, G2)
            pltpu.async_copy(pbuf0, acc_sh.at[dkeys.at[c0, 0]], ssem0, add=True)

            pltpu.make_async_copy(xw_hbm.at[gkeys.at[0, 0]], rows1, gsem1).wait()

            @pl.when(p < CPS // 2 - 1)
            def _prefetch():
                pltpu.async_copy(xw_hbm.at[gkeys.at[c0 + 2, 0]], rows0, gsem0)

            @pl.when(p > 0)
            def _drain_s1():
                pltpu.make_async_copy(
                    pbuf1, acc_sh.at[dkeys.at[0, 0]], ssem1).wait()

            _scale_pack(rows1, pbuf1, (c0 + 1) * G2, G2)
            pltpu.async_copy(pbuf1, acc_sh.at[dkeys.at[c0 + 1, 0]], ssem1,
                             add=True)
            return inner

        lax.fori_loop(0, CPS // 2, pairbody, 0)
        pltpu.make_async_copy(pbuf0, acc_sh.at[dkeys.at[0, 0]], ssem0).wait()
        pltpu.make_async_copy(pbuf1, acc_sh.at[dkeys.at[0, 0]], ssem1).wait()
        return carry

    lax.fori_loop(0, NSCH, superchunk, 0)

    off = base + NSCH * SCH
    pltpu.sync_copy(src_hbm.at[pl.ds(off, TAIL)], srcm.at[pl.ds(0, TAIL)])
    pltpu.sync_copy(dst_hbm.at[pl.ds(off, TAIL)], dstm.at[pl.ds(0, TAIL)])
    pltpu.sync_copy(et_hbm.at[pl.ds(off, TAIL)], etm.at[pl.ds(0, TAIL)])
    pltpu.sync_copy(s_hbm.at[pl.ds(off, TAIL)], sm.at[pl.ds(0, TAIL)])
    for j in range(TAIL // L):
        s16 = srcm[pl.ds(j * L, L)]
        t16 = etm[pl.ds(j * L, L)]
        d16 = dstm[pl.ds(j * L, L)]
        gidx_t[pl.ds(j * L, L)] = t16 * N_NODES + s16
        didx_t[pl.ds(j * L, L)] = d16 >> 1
    pltpu.async_copy(xw_hbm.at[gidx_t], rows0.at[pl.ds(0, TAIL)], gsem0).wait()
    _scale_pack(rows0, pbuf0, 0, TAIL)
    pltpu.sync_copy(pbuf0.at[pl.ds(0, TAIL)], acc_sh.at[didx_t], add=True)

    plsc.subcore_barrier()
    for k, nrows in _COPY_CHUNKS:
        pltpu.sync_copy(acc_sh.at[pl.ds(row0 + k, nrows)], pbuf0.at[pl.ds(0, nrows)])
        pltpu.sync_copy(pbuf0.at[pl.ds(0, nrows)],
                        out_hbm.at[cid, pl.ds(row0 + k, nrows)])

    @pl.when(sid == 0)
    def _write_rem():
        pltpu.sync_copy(acc_sh.at[pl.ds(NS * RPS, REM)], pbuf0.at[pl.ds(0, REM)])
        pltpu.sync_copy(pbuf0.at[pl.ds(0, REM)],
                        out_hbm.at[cid, pl.ds(NS * RPS, REM)])


def _fin_body(hp_ref, x_ref, lw_ref, b_ref, o_ref):
    o_ref[...] = (jnp.concatenate([hp_ref[0], hp_ref[1]], axis=1)
                  + jnp.dot(x_ref[...], lw_ref[...],
                            preferred_element_type=jnp.float32)
                  + b_ref[...])


def _stage_e(hp, x, loop_weight, h_bias):
    return pl.pallas_call(
        _fin_body,
        out_shape=jax.ShapeDtypeStruct((N_NODES, D), jnp.float32),
    )(hp, x, loop_weight, h_bias.reshape(1, D))


@jax.jit
def kernel(x, edge_index, edge_type, coeff, bases, h_bias, loop_weight):
    src = edge_index[0].astype(jnp.int32)
    dst = edge_index[1].astype(jnp.int32)
    et = edge_type.astype(jnp.int32)

    xw = _stage_a(coeff, bases, x).reshape(RN, D)
    deg_all = _deg_kernel(dst, et)
    inv_deg = _stage_c(deg_all)
    scales = _scale_kernel(inv_deg, dst, et)
    hp = _scatter_kernel(xw, scales, src, dst, et)
    return _stage_e(hp.reshape(NC, N_NODES, DH), x, loop_weight, h_bias)


# speedup vs baseline: 11.5641x; 1.0321x over previous
import functools

import jax
import jax.numpy as jnp
from jax import lax
from jax.experimental import pallas as pl
from jax.experimental.pallas import tpu as pltpu
from jax.experimental.pallas import tpu_sc as plsc

N_NODES = 10000
N_EDGES = 320000
N_REL = 8
N_BASES = 4
D = 128
DH = D // 2
RN = N_REL * N_NODES

NC, NS, L = 2, 16, 16
NW = NC * NS
EPW = N_EDGES // NW
EPT = N_EDGES // NS
G2 = 64
SCH = 1536
CPS = SCH // G2
NSCH = 13
TAIL = EPT - NSCH * SCH
ECH = 10000
NPAIR = N_NODES // 2
RPS = 312
REM = NPAIR - NS * RPS
_COPY_CHUNKS = ((0, 64), (64, 64), (128, 64), (192, 64), (256, 56))

_mesh = plsc.VectorSubcoreMesh(
    core_axis_name="c", subcore_axis_name="s", num_cores=NC, num_subcores=NS
)
_sc_params = pltpu.CompilerParams(needs_layout_passes=False)


def _xw_body(coeff_ref, bases_ref, x_ref, o_ref):
    r = pl.program_id(0)
    w = coeff_ref[r, 0] * bases_ref[0]
    for b in range(1, N_BASES):
        w = w + coeff_ref[r, b] * bases_ref[b]
    o_ref[0] = jnp.dot(x_ref[...], w, preferred_element_type=jnp.float32)


def _stage_a(coeff, bases, x):
    return pl.pallas_call(
        _xw_body,
        grid=(N_REL,),
        in_specs=[
            pl.BlockSpec(memory_space=pltpu.SMEM),
            pl.BlockSpec((N_BASES, D, D), lambda r: (0, 0, 0)),
            pl.BlockSpec((N_NODES, D), lambda r: (0, 0)),
        ],
        out_specs=pl.BlockSpec((1, N_NODES, D), lambda r: (r, 0, 0)),
        out_shape=jax.ShapeDtypeStruct((N_REL, N_NODES, D), jnp.float32),
    )(coeff, bases, x)


@functools.partial(
    pl.kernel,
    out_type=jax.ShapeDtypeStruct((NW * RN,), jnp.float32),
    mesh=_mesh,
    scratch_types=[
        pltpu.VMEM((RN,), jnp.float32),
        pltpu.VMEM((ECH,), jnp.int32),
        pltpu.VMEM((ECH,), jnp.int32),
    ],
    compiler_params=_sc_params,
)
def _deg_kernel(dst_hbm, et_hbm, out_hbm, deg_v, d_buf, t_buf):
    cid = lax.axis_index("c")
    sid = lax.axis_index("s")
    wid = sid * NC + cid
    base = wid * EPW

    zero16 = jnp.zeros((L,), jnp.float32)

    def zbody(i, carry):
        for u in range(8):
            deg_v[pl.ds(i * 128 + u * L, L)] = zero16
        return carry

    lax.fori_loop(0, RN // 128, zbody, 0)

    one16 = jnp.full((L,), 1.0, jnp.float32)

    def cbody(c, carry):
        off = base + c * ECH
        pltpu.sync_copy(dst_hbm.at[pl.ds(off, ECH)], d_buf)
        pltpu.sync_copy(et_hbm.at[pl.ds(off, ECH)], t_buf)

        def ebody(j, inner):
            d16 = d_buf[pl.ds(j * L, L)]
            t16 = t_buf[pl.ds(j * L, L)]
            plsc.addupdate_scatter(deg_v, [t16 * N_NODES + d16], one16)
            return inner

        lax.fori_loop(0, ECH // L, ebody, 0)
        return carry

    lax.fori_loop(0, EPW // ECH, cbody, 0)
    pltpu.sync_copy(deg_v, out_hbm.at[pl.ds(wid * RN, RN)])


def _inv_body(d_ref, o_ref):
    s = jnp.sum(d_ref[...], axis=0)
    o_ref[...] = 1.0 / jnp.maximum(s, 1.0)


def _stage_c(deg_all):
    d3 = deg_all.reshape(NW, RN // D, D)
    out = pl.pallas_call(
        _inv_body,
        out_shape=jax.ShapeDtypeStruct((RN // D, D), jnp.float32),
    )(d3)
    return out.reshape(RN)


@functools.partial(
    pl.kernel,
    out_type=jax.ShapeDtypeStruct((N_EDGES,), jnp.float32),
    mesh=_mesh,
    scratch_types=[
        pltpu.VMEM((RN,), jnp.float32),
        pltpu.VMEM((ECH,), jnp.int32),
        pltpu.VMEM((ECH,), jnp.int32),
        pltpu.VMEM((ECH,), jnp.float32),
    ],
    compiler_params=_sc_params,
)
def _scale_kernel(inv_hbm, dst_hbm, et_hbm, out_hbm, inv_v, d_buf, t_buf,
                  s_buf):
    cid = lax.axis_index("c")
    sid = lax.axis_index("s")
    wid = sid * NC + cid
    base = wid * EPW

    pltpu.sync_copy(inv_hbm, inv_v)

    def cbody(c, carry):
        off = base + c * ECH
        pltpu.sync_copy(dst_hbm.at[pl.ds(off, ECH)], d_buf)
        pltpu.sync_copy(et_hbm.at[pl.ds(off, ECH)], t_buf)

        @plsc.parallel_loop(0, ECH // L)
        def ebody(j):
            d16 = d_buf[pl.ds(j * L, L)]
            t16 = t_buf[pl.ds(j * L, L)]
            s_buf[pl.ds(j * L, L)] = plsc.load_gather(
                inv_v, [t16 * N_NODES + d16])

        pltpu.sync_copy(s_buf, out_hbm.at[pl.ds(off, ECH)])
        return carry

    lax.fori_loop(0, EPW // ECH, cbody, 0)


@functools.partial(
    pl.kernel,
    out_type=jax.ShapeDtypeStruct((NC, NPAIR, D), jnp.float32),
    mesh=_mesh,
    scratch_types=[
        pltpu.VMEM_SHARED((NPAIR, D), jnp.float32),
        pltpu.VMEM((G2, D), jnp.float32),
        pltpu.VMEM((G2, D), jnp.float32),
        pltpu.VMEM((G2, D), jnp.float32),
        pltpu.VMEM((G2, D), jnp.float32),
        pltpu.VMEM((SCH,), jnp.int32),
        pltpu.VMEM((SCH,), jnp.int32),
        pltpu.VMEM((SCH,), jnp.int32),
        pltpu.VMEM((SCH,), jnp.float32),
        pltpu.VMEM((CPS, 1, G2), jnp.int32),
        pltpu.VMEM((CPS, 1, G2), jnp.int32),
        pltpu.VMEM((TAIL,), jnp.int32),
        pltpu.VMEM((TAIL,), jnp.int32),
        pltpu.SemaphoreType.DMA,
        pltpu.SemaphoreType.DMA,
        pltpu.SemaphoreType.DMA,
        pltpu.SemaphoreType.DMA,
        pltpu.SemaphoreType.DMA,
    ],
    compiler_params=_sc_params,
)
def _scatter_kernel(xw_hbm, s_hbm, src_hbm, dst_hbm, et_hbm, out_hbm,
                    acc_sh, rows0, rows1, pbuf0, pbuf1,
                    srcm, dstm, etm, sm, gkeys, dkeys, gidx_t, didx_t,
                    gsem0, gsem1, ssem0, ssem1, msem):
    cid = lax.axis_index("c")
    sid = lax.axis_index("s")
    base = sid * EPT
    col0 = cid * DH

    zero16 = jnp.zeros((L,), jnp.float32)

    def zb(i, carry):
        for u in range(D // L):
            pbuf0[i, pl.ds(u * L, L)] = zero16
        return carry

    lax.fori_loop(0, G2, zb, 0)
    row0 = sid * RPS
    for k, nrows in _COPY_CHUNKS:
        pltpu.sync_copy(pbuf0.at[pl.ds(0, nrows)], acc_sh.at[pl.ds(row0 + k, nrows)])

    @pl.when(sid == 0)
    def _zero_rem():
        pltpu.sync_copy(pbuf0.at[pl.ds(0, REM)], acc_sh.at[pl.ds(NS * RPS, REM)])

    plsc.subcore_barrier()

    def _scale_pack(rows_ref, p_ref, eoff, n_edges):
        @plsc.parallel_loop(0, n_edges // L)
        def mbody(mb):
            o = eoff + mb * L
            s16 = sm[pl.ds(o, L)]
            par16 = (dstm[pl.ds(o, L)] & 1) * DH
            for mm in range(L):
                m = mb * L + mm
                sv = s16[mm]
                pv = par16[mm]
                nv = DH - pv
                for q in range(DH // L):
                    p_ref[m, pl.ds(pv + q * L, L)] = (
                        rows_ref[m, pl.ds(col0 + q * L, L)] * sv)
                    p_ref[m, pl.ds(nv + q * L, L)] = zero16

    def keys_body(c2, carry):
        for j in range(G2 // L):
            o = c2 * G2 + j * L
            s16 = srcm[pl.ds(o, L)]
            t16 = etm[pl.ds(o, L)]
            d16 = dstm[pl.ds(o, L)]
            gkeys[c2, 0, pl.ds(j * L, L)] = t16 * N_NODES + s16
            dkeys[c2, 0, pl.ds(j * L, L)] = d16 >> 1
        return carry

    def superchunk(sc, carry):
        off = base + sc * SCH
        pltpu.async_copy(src_hbm.at[pl.ds(off, SCH)], srcm, msem)
        pltpu.async_copy(dst_hbm.at[pl.ds(off, SCH)], dstm, msem)
        pltpu.async_copy(et_hbm.at[pl.ds(off, SCH)], etm, msem)
        pltpu.async_copy(s_hbm.at[pl.ds(off, SCH)], sm, msem)
        pltpu.make_async_copy(src_hbm.at[pl.ds(off, SCH)], srcm, msem).wait()
        pltpu.make_async_copy(dst_hbm.at[pl.ds(off, SCH)], dstm, msem).wait()
        pltpu.make_async_copy(et_hbm.at[pl.ds(off, SCH)], etm, msem).wait()
        pltpu.make_async_copy(s_hbm.at[pl.ds(off, SCH)], sm, msem).wait()
        lax.fori_loop(0, CPS, keys_body, 0)

        pltpu.async_copy(xw_hbm.at[gkeys.at[0, 0]], rows0, gsem0)

        def pairbody(p, inner):
            c0 = 2 * p
            pltpu.make_async_copy(xw_hbm.at[gkeys.at[0, 0]], rows0, gsem0).wait()
            pltpu.async_copy(xw_hbm.at[gkeys.at[c0 + 1, 0]], rows1, gsem1)

            @pl.when(p > 0)
            def _drain_s0():
                pltpu.make_async_copy(
                    pbuf0, acc_sh.at[dkeys.at[0, 0]], ssem0).wait()

            _scale_pack(rows0, pbuf0, c0 * G2, G2)
            pltpu.async_copy(pbuf0, acc_sh.at[dkeys.at[c0, 0]], ssem0, add=True)

            pltpu.make_async_copy(xw_hbm.at[gkeys.at[0, 0]], rows1, gsem1).wait()

            @pl.when(p < CPS // 2 - 1)
            def _prefetch():
                pltpu.async_copy(xw_hbm.at[gkeys.at[c0 + 2, 0]], rows0, gsem0)

            @pl.when(p > 0)
            def _drain_s1():
                pltpu.make_async_copy(
                    pbuf1, acc_sh.at[dkeys.at[0, 0]], ssem1).wait()

            _scale_pack(rows1, pbuf1, (c0 + 1) * G2, G2)
            pltpu.async_copy(pbuf1, acc_sh.at[dkeys.at[c0 + 1, 0]], ssem1,
                             add=True)
            return inner

        lax.fori_loop(0, CPS // 2, pairbody, 0)
        pltpu.make_async_copy(pbuf0, acc_sh.at[dkeys.at[0, 0]], ssem0).wait()
        pltpu.make_async_copy(pbuf1, acc_sh.at[dkeys.at[0, 0]], ssem1).wait()
        return carry

    lax.fori_loop(0, NSCH, superchunk, 0)

    off = base + NSCH * SCH
    pltpu.sync_copy(src_hbm.at[pl.ds(off, TAIL)], srcm.at[pl.ds(0, TAIL)])
    pltpu.sync_copy(dst_hbm.at[pl.ds(off, TAIL)], dstm.at[pl.ds(0, TAIL)])
    pltpu.sync_copy(et_hbm.at[pl.ds(off, TAIL)], etm.at[pl.ds(0, TAIL)])
    pltpu.sync_copy(s_hbm.at[pl.ds(off, TAIL)], sm.at[pl.ds(0, TAIL)])
    for j in range(TAIL // L):
        s16 = srcm[pl.ds(j * L, L)]
        t16 = etm[pl.ds(j * L, L)]
        d16 = dstm[pl.ds(j * L, L)]
        gidx_t[pl.ds(j * L, L)] = t16 * N_NODES + s16
        didx_t[pl.ds(j * L, L)] = d16 >> 1
    pltpu.async_copy(xw_hbm.at[gidx_t], rows0.at[pl.ds(0, TAIL)], gsem0).wait()
    _scale_pack(rows0, pbuf0, 0, TAIL)
    pltpu.sync_copy(pbuf0.at[pl.ds(0, TAIL)], acc_sh.at[didx_t], add=True)

    plsc.subcore_barrier()
    for k, nrows in _COPY_CHUNKS:
        pltpu.sync_copy(acc_sh.at[pl.ds(row0 + k, nrows)], pbuf0.at[pl.ds(0, nrows)])
        pltpu.sync_copy(pbuf0.at[pl.ds(0, nrows)],
                        out_hbm.at[cid, pl.ds(row0 + k, nrows)])

    @pl.when(sid == 0)
    def _write_rem():
        pltpu.sync_copy(acc_sh.at[pl.ds(NS * RPS, REM)], pbuf0.at[pl.ds(0, REM)])
        pltpu.sync_copy(pbuf0.at[pl.ds(0, REM)],
                        out_hbm.at[cid, pl.ds(NS * RPS, REM)])


def _fin_body(hp_ref, x_ref, lw_ref, b_ref, o_ref):
    o_ref[...] = (jnp.concatenate([hp_ref[0], hp_ref[1]], axis=1)
                  + jnp.dot(x_ref[...], lw_ref[...],
                            preferred_element_type=jnp.float32)
                  + b_ref[...])


def _stage_e(hp, x, loop_weight, h_bias):
    return pl.pallas_call(
        _fin_body,
        out_shape=jax.ShapeDtypeStruct((N_NODES, D), jnp.float32),
    )(hp, x, loop_weight, h_bias.reshape(1, D))


@jax.jit
def kernel(x, edge_index, edge_type, coeff, bases, h_bias, loop_weight):
    src = edge_index[0].astype(jnp.int32)
    dst = edge_index[1].astype(jnp.int32)
    et = edge_type.astype(jnp.int32)

    xw = _stage_a(coeff, bases, x).reshape(RN, D)
    deg_all = _deg_kernel(dst, et)
    inv_deg = _stage_c(deg_all)
    scales = _scale_kernel(inv_deg, dst, et)
    hp = _scatter_kernel(xw, scales, src, dst, et)
    return _stage_e(hp.reshape(NC, N_NODES, DH), x, loop_weight, h_bias)
